# baseline (speedup 1.0000x reference)
import jax
import jax.numpy as jnp
from jax.experimental import pallas as pl
from jax.experimental.pallas import tpu as pltpu

L = 3
UNITS = 1024
IN_DIM = 1024
BATCH = 16
N_ITER = 5
UT = 256
T = UNITS // UT
WROWS = 2 * UNITS + 8


def _body(k1, k1n, k2, k3, k3n, k4, TA0, TI0, Cinhib0, masks,
          x0, gain2, k6b, kdt1, cp_out, wcomb, vscr):
    l = pl.program_id(0)
    t = pl.program_id(1)

    m = masks[0]
    ka = jnp.where(m > 0, k1[0] / (k1n[0] + k2[0]) * TA0[0], 0.0)
    ki = jnp.where(m < 0, k3[0] / (k3n[0] + k4[0]) * TI0[0], 0.0)
    wcomb[l, pl.ds(t * UT, UT), :] = k2[0] * ka
    wcomb[l, pl.ds(UNITS + t * UT, UT), :] = Cinhib0[0] * ki
    part = jnp.sum(ka + ki, axis=0, keepdims=True)

    @pl.when(t == 0)
    def _():
        vscr[l] = part

    @pl.when(t != 0)
    def _():
        vscr[l] = vscr[l] + part

    @pl.when(jnp.logical_and(l == L - 1, t == T - 1))
    def _():
        zeros8 = jnp.zeros((8, IN_DIM), dtype=jnp.float32)
        for ll in range(L):
            wcomb[ll, pl.ds(2 * UNITS, 8), :] = zeros8
            wcomb[ll, 2 * UNITS, :] = vscr[ll][0]
        X0 = x0[...]
        cp = jnp.ones((BATCH, 1), dtype=jnp.float32)
        for _ in range(N_ITER):
            new_cp = jnp.ones_like(cp)
            X = X0
            for ll in range(L):
                y = jax.lax.dot_general(
                    X, wcomb[ll], (((1,), (1,)), ((), ())),
                    preferred_element_type=jnp.float32)
                s = y[:, 2 * UNITS:2 * UNITS + 1]
                new_cp = new_cp + s / cp
                num = y[:, :UNITS] * (gain2[ll] * cp)
                den = kdt1[ll] * (cp * cp) + k6b[ll] * y[:, UNITS:2 * UNITS]
                X = num / den
            cp = new_cp
        cp_out[...] = cp


def kernel(inputs, k1, k1n, k2, k3, k3n, k4, k5, k5n, k6, kdI, kdT,
           TA0, TI0, Cinhib0, masks, E0):
    f32 = jnp.float32

    gain2 = (k5 / (k5 + k5n) * E0).reshape(L, 1, UNITS)
    k6b = (k6 * E0 / (kdI + 1e-6)).reshape(L, 1, UNITS)
    kdt1 = (kdT + 1e-6).reshape(L, 1, UNITS)

    mat = lambda: pl.BlockSpec((1, UT, IN_DIM), lambda l, t: (l, t, 0))
    vec = lambda: pl.BlockSpec((L, 1, UNITS), lambda l, t: (0, 0, 0))

    cp = pl.pallas_call(
        _body,
        grid=(L, T),
        in_specs=[mat() for _ in range(10)] + [
            pl.BlockSpec((BATCH, IN_DIM), lambda l, t: (0, 0)),
            vec(), vec(), vec(),
        ],
        out_specs=pl.BlockSpec((BATCH, 1), lambda l, t: (0, 0)),
        out_shape=jax.ShapeDtypeStruct((BATCH, 1), f32),
        scratch_shapes=[
            pltpu.VMEM((L, WROWS, IN_DIM), f32),
            pltpu.VMEM((L, 1, IN_DIM), f32),
        ],
    )(k1, k1n, k2, k3, k3n, k4, TA0, TI0, Cinhib0, masks,
      inputs, gain2, k6b, kdt1)
    return cp


# speedup vs baseline: 1.0073x; 1.0073x over previous
import jax
import jax.numpy as jnp
from jax.experimental import pallas as pl
from jax.experimental.pallas import tpu as pltpu

L = 3
UNITS = 1024
IN_DIM = 1024
BATCH = 16
N_ITER = 5
UT = 256
T = UNITS // UT


def _body(k1a, k1na, k2a, k3a, k3na, k4a, TA0a, TI0a, Cinhib0a, masksa,
          k1b, k1nb, k2b, k3b, k3nb, k4b, TA0b, TI0b, Cinhib0b, masksb,
          x0, gain2, k6b, kdt1, cp_out, wcomb, vscr):
    l = pl.program_id(0)
    t = pl.program_id(1)

    H = UT // 2
    ma = masksa[0]
    kaa = jnp.where(ma > 0, k1a[0] / (k1na[0] + k2a[0]) * TA0a[0], 0.0)
    kia = jnp.where(ma < 0, k3a[0] / (k3na[0] + k4a[0]) * TI0a[0], 0.0)
    wcomb[l, pl.ds(t * UT, H), :] = k2a[0] * kaa
    wcomb[l, pl.ds(UNITS + t * UT, H), :] = Cinhib0a[0] * kia
    mb = masksb[0]
    kab = jnp.where(mb > 0, k1b[0] / (k1nb[0] + k2b[0]) * TA0b[0], 0.0)
    kib = jnp.where(mb < 0, k3b[0] / (k3nb[0] + k4b[0]) * TI0b[0], 0.0)
    wcomb[l, pl.ds(t * UT + H, H), :] = k2b[0] * kab
    wcomb[l, pl.ds(UNITS + t * UT + H, H), :] = Cinhib0b[0] * kib
    part = jnp.sum(kaa + kia + kab + kib, axis=0, keepdims=True)

    @pl.when(t == 0)
    def _():
        vscr[l] = part

    @pl.when(t != 0)
    def _():
        vscr[l] = vscr[l] + part

    @pl.when(jnp.logical_and(l == L - 1, t == T - 1))
    def _():
        X0 = x0[...]
        cp = jnp.ones((BATCH, 1), dtype=jnp.float32)
        for _ in range(N_ITER):
            new_cp = jnp.ones_like(cp)
            X = X0
            for ll in range(L):
                s = jnp.sum(X * vscr[ll], axis=1, keepdims=True)
                new_cp = new_cp + s / cp
                y = jax.lax.dot_general(
                    X, wcomb[ll], (((1,), (1,)), ((), ())),
                    preferred_element_type=jnp.float32)
                act = y[:, :UNITS] * gain2[ll] / cp
                denom = kdt1[ll] + k6b[ll] * y[:, UNITS:] / (cp * cp)
                X = act / denom
            cp = new_cp
        cp_out[...] = cp


def kernel(inputs, k1, k1n, k2, k3, k3n, k4, k5, k5n, k6, kdI, kdT,
           TA0, TI0, Cinhib0, masks, E0):
    f32 = jnp.float32

    gain2 = (k5 / (k5 + k5n) * E0).reshape(L, 1, UNITS)
    k6b = (k6 * E0 / (kdI + 1e-6)).reshape(L, 1, UNITS)
    kdt1 = (kdT + 1e-6).reshape(L, 1, UNITS)

    H = UT // 2
    mata = lambda: pl.BlockSpec((1, H, IN_DIM), lambda l, t: (l, 2 * t, 0))
    matb = lambda: pl.BlockSpec((1, H, IN_DIM), lambda l, t: (l, 2 * t + 1, 0))
    vec = lambda: pl.BlockSpec((L, 1, UNITS), lambda l, t: (0, 0, 0))

    cp = pl.pallas_call(
        _body,
        grid=(L, T),
        in_specs=[mata() for _ in range(10)] + [matb() for _ in range(10)] + [
            pl.BlockSpec((BATCH, IN_DIM), lambda l, t: (0, 0)),
            vec(), vec(), vec(),
        ],
        out_specs=pl.BlockSpec((BATCH, 1), lambda l, t: (0, 0)),
        out_shape=jax.ShapeDtypeStruct((BATCH, 1), f32),
        scratch_shapes=[
            pltpu.VMEM((L, 2 * UNITS, IN_DIM), f32),
            pltpu.VMEM((L, 1, IN_DIM), f32),
        ],
    )(k1, k1n, k2, k3, k3n, k4, TA0, TI0, Cinhib0, masks,
      k1, k1n, k2, k3, k3n, k4, TA0, TI0, Cinhib0, masks,
      inputs, gain2, k6b, kdt1)
    return cp
